# baseline (speedup 1.0000x reference)
import functools

import jax
import jax.numpy as jnp
from jax import lax
from jax.experimental import pallas as pl
from jax.experimental.pallas import tpu as pltpu
from jax.experimental.pallas import tpu_sc as plsc

D = 64
DP = 128
DPP = 129
NW = 32
CHUNK = 128
NG = CHUNK // 16
NBUF = 2
NOUT = 1
UNROLL = 2


def _transpose_block(src, dst, rows_j, par_j):
    lag = 8
    pending = []
    for d in range(D):
        for j in range(NG):
            vals = plsc.load_gather(src, [rows_j[j], par_j[j] + d])
            pending.append((d, j, vals))
            if len(pending) > lag:
                dd, jj, v = pending.pop(0)
                dst[dd, pl.ds(16 * jj, 16)] = v
    for dd, jj, v in pending:
        dst[dd, pl.ds(16 * jj, 16)] = v


def _build_gather(seq: int, batch: int):
    nb = batch // CHUNK
    assert nb == NW
    nloop = seq // UNROLL
    assert seq == nloop * UNROLL
    mesh = plsc.VectorSubcoreMesh(core_axis_name="c", subcore_axis_name="s")

    @functools.partial(
        pl.kernel,
        mesh=mesh,
        compiler_params=pltpu.CompilerParams(needs_layout_passes=False),
        out_type=jax.ShapeDtypeStruct((seq, D, batch), jnp.float32),
        scratch_types=[
            pltpu.VMEM((seq, CHUNK), jnp.int32),
            pltpu.VMEM((seq, CHUNK), jnp.int32),
            [pltpu.VMEM((CHUNK, DPP), jnp.float32) for _ in range(NBUF)],
            [pltpu.VMEM((D, CHUNK), jnp.float32) for _ in range(NOUT)],
            [pltpu.SemaphoreType.DMA for _ in range(NBUF)],
            [pltpu.SemaphoreType.DMA for _ in range(NOUT)],
        ],
    )
    def emb(table_hbm, idxhi_hbm, idxlo_hbm, out_hbm,
            idx_v, par_v, bufs, bufts, gsems, psems):
        wid = lax.axis_index("s") * 2 + lax.axis_index("c")
        col = wid * CHUNK
        iota16 = lax.iota(jnp.int32, 16)
        rows_j = [iota16 + 16 * j for j in range(NG)]
        pltpu.sync_copy(idxhi_hbm.at[wid], idx_v)
        pltpu.sync_copy(idxlo_hbm.at[wid], par_v)

        def gather(s, b):
            pltpu.async_copy(table_hbm.at[idx_v.at[s]],
                             bufs[b].at[:, pl.ds(0, DP)], gsems[b])

        def gather_wait(s, b):
            pltpu.make_async_copy(
                table_hbm.at[idx_v.at[s]],
                bufs[b].at[:, pl.ds(0, DP)], gsems[b]).wait()

        def put(s, o):
            pltpu.async_copy(bufts[o], out_hbm.at[s, :, pl.ds(col, CHUNK)],
                             psems[o])

        def put_wait(s, o):
            pltpu.make_async_copy(
                bufts[o], out_hbm.at[s, :, pl.ds(col, CHUNK)], psems[o]).wait()

        def chunk_step(s, k, p, refill):
            b = k % NBUF
            o = k % NOUT
            gather_wait(s, b)
            if k >= NOUT:
                put_wait(s - NOUT, o)
            else:
                @pl.when(p > 0)
                def _():
                    put_wait(s - NOUT, o)
            par_j = [par_v[s, pl.ds(16 * j, 16)] for j in range(NG)]
            _transpose_block(bufs[b], bufts[o], rows_j, par_j)
            if refill == "static":
                gather(s + NBUF, b)
            elif refill == "guarded":
                @pl.when(s + NBUF < seq)
                def _():
                    gather(s + NBUF, b)
            put(s, o)

        for b in range(NBUF):
            gather(b, b)
        def body(p, carry):
            s0 = p * UNROLL
            for k in range(UNROLL):
                chunk_step(s0 + k, k, p, "guarded")
            return carry

        lax.fori_loop(0, nloop, body, 0)
        for o in range(NOUT):
            put_wait(seq - NOUT + o, (seq - NOUT + o + UNROLL) % UNROLL % NOUT)

    return emb


def kernel(input_ids, attention_mask, table):
    b, s = input_ids.shape
    table2 = table.reshape(table.shape[0] // 2, DP)
    ids_w = input_ids.T.reshape(s, NW, CHUNK).transpose(1, 0, 2).astype(jnp.int32)
    ids_hi = ids_w >> 1
    ids_lo = (ids_w & 1) * D
    out3 = _build_gather(s, b)(table2, ids_hi, ids_lo)
    return out3.transpose(2, 0, 1), attention_mask


# speedup vs baseline: 1.0206x; 1.0206x over previous
import functools

import jax
import jax.numpy as jnp
from jax import lax
from jax.experimental import pallas as pl
from jax.experimental.pallas import tpu as pltpu
from jax.experimental.pallas import tpu_sc as plsc

D = 64
DP = 128
NW = 32
CHUNK = 128
NM = D // 16
NBUF = 2
NOUT = 2
UNROLL = 2


def _transpose_block(src, dst, rows_m):
    for i in range(CHUNK):
        cols_i = jnp.full((16,), i, jnp.int32)
        for m in range(NM):
            vals = src[i, pl.ds(16 * m, 16)]
            plsc.store_scatter(dst, [rows_m[m], cols_i], vals)


def _build_gather(seq: int, batch: int):
    nb = batch // CHUNK
    assert nb == NW
    nloop = seq // UNROLL
    assert seq == nloop * UNROLL
    mesh = plsc.VectorSubcoreMesh(core_axis_name="c", subcore_axis_name="s")

    @functools.partial(
        pl.kernel,
        mesh=mesh,
        compiler_params=pltpu.CompilerParams(needs_layout_passes=False),
        out_type=jax.ShapeDtypeStruct((seq, D, batch), jnp.float32),
        scratch_types=[
            pltpu.VMEM((seq, CHUNK), jnp.int32),
            [pltpu.VMEM((CHUNK, DP), jnp.float32) for _ in range(NBUF)],
            [pltpu.VMEM((D, CHUNK), jnp.float32) for _ in range(NOUT)],
            [pltpu.SemaphoreType.DMA for _ in range(NBUF)],
            [pltpu.SemaphoreType.DMA for _ in range(NOUT)],
        ],
    )
    def emb(table_hbm, idx_hbm, out_hbm, idx_v, bufs, bufts, gsems, psems):
        wid = lax.axis_index("s") * 2 + lax.axis_index("c")
        col = wid * CHUNK
        iota16 = lax.iota(jnp.int32, 16)
        rows_m = [iota16 + 16 * m for m in range(NM)]
        pltpu.sync_copy(idx_hbm.at[wid], idx_v)

        def gather(s, b):
            pltpu.async_copy(table_hbm.at[idx_v.at[s]], bufs[b], gsems[b])

        def gather_wait(s, b):
            pltpu.make_async_copy(
                table_hbm.at[idx_v.at[s]], bufs[b], gsems[b]).wait()

        def put(s, o):
            pltpu.async_copy(bufts[o], out_hbm.at[s, :, pl.ds(col, CHUNK)],
                             psems[o])

        def put_wait(s, o):
            pltpu.make_async_copy(
                bufts[o], out_hbm.at[s, :, pl.ds(col, CHUNK)], psems[o]).wait()

        def chunk_step(s, k, p):
            b = k % NBUF
            o = k % NOUT
            gather_wait(s, b)

            @pl.when(p > 0)
            def _():
                put_wait(s - NOUT, o)
            _transpose_block(bufs[b], bufts[o], rows_m)
            @pl.when(s + NBUF < seq)
            def _():
                gather(s + NBUF, b)
            put(s, o)

        for b in range(NBUF):
            gather(b, b)

        def body(p, carry):
            s0 = p * UNROLL
            for k in range(UNROLL):
                chunk_step(s0 + k, k, p)
            return carry

        lax.fori_loop(0, nloop, body, 0)
        for o in range(NOUT):
            put_wait(seq - NOUT + o, o)

    return emb


def kernel(input_ids, attention_mask, table):
    b, s = input_ids.shape
    table_pad = jnp.pad(table, ((0, 0), (0, DP - D)))
    ids_w = input_ids.T.reshape(s, NW, CHUNK).transpose(1, 0, 2).astype(jnp.int32)
    out3 = _build_gather(s, b)(table_pad, ids_w)
    return out3.transpose(2, 0, 1), attention_mask


# speedup vs baseline: 1.5790x; 1.5470x over previous
import functools

import jax
import jax.numpy as jnp
from jax import lax
from jax.experimental import pallas as pl
from jax.experimental.pallas import tpu as pltpu
from jax.experimental.pallas import tpu_sc as plsc

D = 64
DP = 128
NW = 32
CHUNK = 128
NM = D // 16
NBUF = 2
NOUT = 2
UNROLL = 2


def _build_gather(seq: int, batch: int):
    nb = batch // CHUNK
    assert nb == NW
    nloop = seq // UNROLL
    assert seq == nloop * UNROLL
    mesh = plsc.VectorSubcoreMesh(core_axis_name="c", subcore_axis_name="s")

    @functools.partial(
        pl.kernel,
        mesh=mesh,
        compiler_params=pltpu.CompilerParams(needs_layout_passes=False),
        out_type=jax.ShapeDtypeStruct((seq * batch, DP), jnp.float32),
        scratch_types=[
            pltpu.VMEM((seq, CHUNK), jnp.int32),
            [pltpu.VMEM((CHUNK, DP), jnp.float32) for _ in range(NBUF)],
            [pltpu.SemaphoreType.DMA for _ in range(NBUF)],
            [pltpu.SemaphoreType.DMA for _ in range(NOUT)],
        ],
    )
    def emb(table_hbm, idx_hbm, out_hbm, idx_v, bufs, gsems, psems):
        wid = lax.axis_index("s") * 2 + lax.axis_index("c")
        base = wid * seq * CHUNK
        pltpu.sync_copy(idx_hbm.at[wid], idx_v)

        def gather(s, b):
            pltpu.async_copy(table_hbm.at[idx_v.at[s]], bufs[b], gsems[b])

        def gather_wait(s, b):
            pltpu.make_async_copy(
                table_hbm.at[idx_v.at[s]], bufs[b], gsems[b]).wait()

        def put(s, b, o):
            pltpu.async_copy(bufs[b],
                             out_hbm.at[pl.ds(base + s * CHUNK, CHUNK)],
                             psems[o])

        def put_wait(s, b, o):
            pltpu.make_async_copy(
                bufs[b],
                out_hbm.at[pl.ds(base + s * CHUNK, CHUNK)], psems[o]).wait()

        def chunk_step(s, k, p):
            b = k % NBUF
            o = k % NOUT
            gather_wait(s, b)

            @pl.when(p > 0)
            def _():
                put_wait(s - NOUT, (k + NBUF - NOUT) % NBUF, o)
            put(s, b, o)
            @pl.when(s + NBUF < seq)
            def _():
                gather(s + NBUF, b)

        for b in range(NBUF):
            gather(b, b)

        def body(p, carry):
            s0 = p * UNROLL
            for k in range(UNROLL):
                chunk_step(s0 + k, k, p)
            return carry

        lax.fori_loop(0, nloop, body, 0)
        for o in range(NOUT):
            s = seq - NOUT + o
            put_wait(s, s % NBUF, o)

    return emb


def kernel(input_ids, attention_mask, table):
    b, s = input_ids.shape
    table_pad = jnp.pad(table, ((0, 0), (0, DP - D)))
    ids_w = input_ids.reshape(NW, (b * s) // (NW * CHUNK), CHUNK).astype(jnp.int32)
    out2 = _build_gather(s, b)(table_pad, ids_w)
    return out2[:, :D].reshape(b, s, D), attention_mask


# speedup vs baseline: 1.5826x; 1.0023x over previous
import functools

import jax
import jax.numpy as jnp
from jax import lax
from jax.experimental import pallas as pl
from jax.experimental.pallas import tpu as pltpu
from jax.experimental.pallas import tpu_sc as plsc

D = 64
DP = 128
NW = 32
CHUNK = 128
NM = D // 16
NBUF = 3
UNROLL = 3


def _build_gather(seq: int, batch: int):
    nb = batch // CHUNK
    assert nb == NW
    nloop = seq // UNROLL
    mesh = plsc.VectorSubcoreMesh(core_axis_name="c", subcore_axis_name="s")

    @functools.partial(
        pl.kernel,
        mesh=mesh,
        compiler_params=pltpu.CompilerParams(needs_layout_passes=False),
        out_type=jax.ShapeDtypeStruct((seq * batch, DP), jnp.float32),
        scratch_types=[
            pltpu.VMEM((seq, CHUNK), jnp.int32),
            [pltpu.VMEM((CHUNK, DP), jnp.float32) for _ in range(NBUF)],
            [pltpu.SemaphoreType.DMA for _ in range(NBUF)],
            [pltpu.SemaphoreType.DMA for _ in range(NBUF)],
        ],
    )
    def emb(table_hbm, idx_hbm, out_hbm, idx_v, bufs, gsems, psems):
        wid = lax.axis_index("s") * 2 + lax.axis_index("c")
        base = wid * seq * CHUNK
        pltpu.sync_copy(idx_hbm.at[wid], idx_v)

        def gather(s, b):
            pltpu.async_copy(table_hbm.at[idx_v.at[s]], bufs[b], gsems[b])

        def gather_wait(s, b):
            pltpu.make_async_copy(
                table_hbm.at[idx_v.at[s]], bufs[b], gsems[b]).wait()

        def put(s, b):
            pltpu.async_copy(bufs[b],
                             out_hbm.at[pl.ds(base + s * CHUNK, CHUNK)],
                             psems[b])

        def put_wait(s, b):
            pltpu.make_async_copy(
                bufs[b],
                out_hbm.at[pl.ds(base + s * CHUNK, CHUNK)], psems[b]).wait()

        def chunk_step(s, k, first, refill):
            b = k % NBUF
            bp = (k + NBUF - 1) % NBUF
            gather_wait(s, b)
            put(s, b)
            if not first:
                put_wait(s - 1, bp)
            if refill == "static":
                gather(s + 2, (k + 2) % NBUF)
            elif refill == "guarded":
                @pl.when(s + 2 < seq)
                def _():
                    gather(s + 2, (k + 2) % NBUF)

        for b in range(2):
            gather(b, b)

        def body(p, carry):
            s0 = p * UNROLL
            for k in range(UNROLL):
                chunk_step(s0 + k, k, False, "guarded")
            return carry

        for k in range(UNROLL):
            chunk_step(k, k, k == 0, "static")
        lax.fori_loop(1, nloop, body, 0)
        tail0 = nloop * UNROLL
        for t in range(seq - tail0):
            chunk_step(tail0 + t, t, False, "none")
        put_wait(seq - 1, (seq - 1) % NBUF)

    return emb


def kernel(input_ids, attention_mask, table):
    b, s = input_ids.shape
    table_pad = jnp.pad(table, ((0, 0), (0, DP - D)))
    ids_w = input_ids.reshape(NW, (b * s) // (NW * CHUNK), CHUNK).astype(jnp.int32)
    out2 = _build_gather(s, b)(table_pad, ids_w)
    return out2[:, :D].reshape(b, s, D), attention_mask
